# baseline (speedup 1.0000x reference)
import jax
import jax.numpy as jnp
from jax.experimental import pallas as pl
from jax.experimental.pallas import tpu as pltpu

NUM_EMBED_K = 100000
EMB_DIM_K = 768
TOPK_K = 8
N_TOKENS_K = 64

BLOCK_ROWS = 2048
NBLOCKS = (NUM_EMBED_K + BLOCK_ROWS - 1) // BLOCK_ROWS
PAD_COLS = NBLOCKS * BLOCK_ROWS
TAIL_VALID = NUM_EMBED_K - (NBLOCKS - 1) * BLOCK_ROWS


GRID_STEPS = 25


def _topk_kernel(x_ref, wa_ref, wb_ref, vals_ref, idx_ref, logit_ref):
    i = pl.program_id(0)
    nsteps = pl.num_programs(0)

    x = x_ref[...]
    la = jax.lax.dot_general(
        wa_ref[...], x, (((1,), (1,)), ((), ())),
        preferred_element_type=jnp.float32,
    )
    logit_ref[i] = la.T

    @pl.when(i < GRID_STEPS - 1)
    def _do_b():
        lb = jax.lax.dot_general(
            wb_ref[...], x, (((1,), (1,)), ((), ())),
            preferred_element_type=jnp.float32,
        )
        logit_ref[GRID_STEPS + i] = lb.T

    @pl.when(i == nsteps - 1)
    def _tail_mask():
        logit_ref[NBLOCKS - 1, :, TAIL_VALID:] = jnp.full(
            (N_TOKENS_K, BLOCK_ROWS - TAIL_VALID), -jnp.inf, jnp.float32
        )

    @pl.when(i == nsteps - 1)
    def _extract():
        BIG = jnp.int32(2**30)
        iota = jax.lax.broadcasted_iota(
            jnp.int32, (N_TOKENS_K, BLOCK_ROWS), 1
        )
        m_out = []
        g_out = []
        gi_prev = jnp.full((N_TOKENS_K, 1), -1, jnp.int32)
        for _ in range(TOPK_K):
            gp = gi_prev

            def body(c, carry, gp=gp):
                M, I = carry
                v = logit_ref[c]
                idx = iota + c * BLOCK_ROWS
                v = jnp.where(idx == gp, -jnp.inf, v)
                logit_ref[c] = v
                for t in range(BLOCK_ROWS // 128):
                    sv = v[:, t * 128:(t + 1) * 128]
                    si = idx[:, t * 128:(t + 1) * 128]
                    upd = (sv > M) | ((sv == M) & (si < I))
                    M = jnp.where(upd, sv, M)
                    I = jnp.where(upd, si, I)
                return M, I

            M0 = jnp.full((N_TOKENS_K, 128), -jnp.inf, jnp.float32)
            I0 = jnp.full((N_TOKENS_K, 128), BIG, jnp.int32)
            M, I = jax.lax.fori_loop(0, NBLOCKS, body, (M0, I0))
            m = jnp.max(M, axis=1, keepdims=True)
            gi = jnp.min(jnp.where(M == m, I, BIG), axis=1, keepdims=True)
            m_out.append(m)
            g_out.append(gi)
            gi_prev = gi

        vals_ref[...] = jnp.concatenate(m_out, axis=1)
        idx_ref[...] = jnp.concatenate(g_out, axis=1)


@jax.jit
def kernel(x, W):
    vals, idx = pl.pallas_call(
        _topk_kernel,
        grid=(GRID_STEPS,),
        in_specs=[
            pl.BlockSpec((N_TOKENS_K, EMB_DIM_K), lambda i: (0, 0)),
            pl.BlockSpec((BLOCK_ROWS, EMB_DIM_K), lambda i: (i, 0)),
            pl.BlockSpec(
                (BLOCK_ROWS, EMB_DIM_K),
                lambda i: (GRID_STEPS + jnp.minimum(i, GRID_STEPS - 2), 0),
            ),
        ],
        out_specs=[
            pl.BlockSpec((N_TOKENS_K, TOPK_K), lambda i: (0, 0)),
            pl.BlockSpec((N_TOKENS_K, TOPK_K), lambda i: (0, 0)),
        ],
        out_shape=[
            jax.ShapeDtypeStruct((N_TOKENS_K, TOPK_K), jnp.float32),
            jax.ShapeDtypeStruct((N_TOKENS_K, TOPK_K), jnp.int32),
        ],
        scratch_shapes=[
            pltpu.VMEM((NBLOCKS, N_TOKENS_K, BLOCK_ROWS), jnp.float32),
        ],
    )(x, W, W)
    return (vals, idx)


# speedup vs baseline: 1.6046x; 1.6046x over previous
import jax
import jax.numpy as jnp
from jax.experimental import pallas as pl
from jax.experimental.pallas import tpu as pltpu

NUM_EMBED_K = 100000
EMB_DIM_K = 768
TOPK_K = 8
N_TOKENS_K = 64

BLOCK_ROWS = 2048
NBLOCKS = (NUM_EMBED_K + BLOCK_ROWS - 1) // BLOCK_ROWS
PAD_COLS = NBLOCKS * BLOCK_ROWS
TAIL_VALID = NUM_EMBED_K - (NBLOCKS - 1) * BLOCK_ROWS
GRID_STEPS = 25
NLEV = 4
BIG_I = 2**30


def _fold_insert(fv_ref, fi_ref, lt, base, mask_invalid):
    L = [fv_ref[k] for k in range(NLEV)]
    J = [fi_ref[k] for k in range(NLEV)]
    iota = jax.lax.broadcasted_iota(jnp.int32, (N_TOKENS_K, 128), 1)
    for t in range(BLOCK_ROWS // 128):
        v = lt[:, t * 128:(t + 1) * 128]
        ix = iota + (base + t * 128)
        if mask_invalid:
            v = jnp.where(ix < NUM_EMBED_K, v, -jnp.inf)
        c = [v > L[k] for k in range(NLEV)]
        nL = [None] * NLEV
        nJ = [None] * NLEV
        for k in range(NLEV - 1, 0, -1):
            nL[k] = jnp.where(c[k], jnp.where(c[k - 1], L[k - 1], v), L[k])
            nJ[k] = jnp.where(c[k], jnp.where(c[k - 1], J[k - 1], ix), J[k])
        nL[0] = jnp.where(c[0], v, L[0])
        nJ[0] = jnp.where(c[0], ix, J[0])
        L, J = nL, nJ
    for k in range(NLEV):
        fv_ref[k] = L[k]
        fi_ref[k] = J[k]


def _topk_kernel(x_ref, wa_ref, wb_ref, vals_ref, idx_ref, logit_ref,
                 fav_ref, fai_ref, fbv_ref, fbi_ref):
    i = pl.program_id(0)
    nsteps = pl.num_programs(0)

    @pl.when(i == 0)
    def _init():
        shape = (NLEV, N_TOKENS_K, 128)
        fav_ref[...] = jnp.full(shape, -jnp.inf, jnp.float32)
        fai_ref[...] = jnp.zeros(shape, jnp.int32)
        fbv_ref[...] = jnp.full(shape, -jnp.inf, jnp.float32)
        fbi_ref[...] = jnp.zeros(shape, jnp.int32)

    x = x_ref[...]
    la = jax.lax.dot_general(
        wa_ref[...], x, (((1,), (1,)), ((), ())),
        preferred_element_type=jnp.float32,
    )
    lat = la.T
    logit_ref[i] = lat
    _fold_insert(fav_ref, fai_ref, lat, i * BLOCK_ROWS, False)

    @pl.when(i < GRID_STEPS - 1)
    def _do_b():
        lb = jax.lax.dot_general(
            wb_ref[...], x, (((1,), (1,)), ((), ())),
            preferred_element_type=jnp.float32,
        )
        lbt = lb.T
        logit_ref[GRID_STEPS + i] = lbt
        _fold_insert(
            fbv_ref, fbi_ref, lbt, (GRID_STEPS + i) * BLOCK_ROWS, True
        )

    @pl.when(i == nsteps - 1)
    def _tail_mask():
        logit_ref[NBLOCKS - 1, :, TAIL_VALID:] = jnp.full(
            (N_TOKENS_K, BLOCK_ROWS - TAIL_VALID), -jnp.inf, jnp.float32
        )

    @pl.when(i == nsteps - 1)
    def _extract():
        ev = jnp.concatenate(
            [fav_ref[k] for k in range(NLEV)]
            + [fbv_ref[k] for k in range(NLEV)], axis=1
        )
        ei = jnp.concatenate(
            [fai_ref[k] for k in range(NLEV)]
            + [fbi_ref[k] for k in range(NLEV)], axis=1
        )
        lev3 = jnp.concatenate([fai_ref[NLEV - 1], fbi_ref[NLEV - 1]], axis=1)
        m_out = []
        g_out = []
        hit3 = jnp.zeros((N_TOKENS_K, 1), jnp.bool_)
        for _ in range(TOPK_K):
            m = jnp.max(ev, axis=1, keepdims=True)
            gi = jnp.min(jnp.where(ev == m, ei, BIG_I), axis=1, keepdims=True)
            m_out.append(m)
            g_out.append(gi)
            hit3 = hit3 | jnp.any(lev3 == gi, axis=1, keepdims=True)
            ev = jnp.where(ei == gi, -jnp.inf, ev)
        vals_ref[...] = jnp.concatenate(m_out, axis=1)
        idx_ref[...] = jnp.concatenate(g_out, axis=1)

        @pl.when(jnp.any(hit3))
        def _fallback():
            iota = jax.lax.broadcasted_iota(
                jnp.int32, (N_TOKENS_K, BLOCK_ROWS), 1
            )
            f_m = []
            f_g = []
            gi_prev = jnp.full((N_TOKENS_K, 1), -1, jnp.int32)
            for _ in range(TOPK_K):
                gp = gi_prev

                def body(c, carry, gp=gp):
                    M, I = carry
                    v = logit_ref[c]
                    idx = iota + c * BLOCK_ROWS
                    v = jnp.where(idx == gp, -jnp.inf, v)
                    logit_ref[c] = v
                    for t in range(BLOCK_ROWS // 128):
                        sv = v[:, t * 128:(t + 1) * 128]
                        si = idx[:, t * 128:(t + 1) * 128]
                        upd = (sv > M) | ((sv == M) & (si < I))
                        M = jnp.where(upd, sv, M)
                        I = jnp.where(upd, si, I)
                    return M, I

                M0 = jnp.full((N_TOKENS_K, 128), -jnp.inf, jnp.float32)
                I0 = jnp.full((N_TOKENS_K, 128), BIG_I, jnp.int32)
                M, I = jax.lax.fori_loop(0, NBLOCKS, body, (M0, I0))
                m = jnp.max(M, axis=1, keepdims=True)
                gi = jnp.min(
                    jnp.where(M == m, I, BIG_I), axis=1, keepdims=True
                )
                f_m.append(m)
                f_g.append(gi)
                gi_prev = gi
            vals_ref[...] = jnp.concatenate(f_m, axis=1)
            idx_ref[...] = jnp.concatenate(f_g, axis=1)


@jax.jit
def kernel(x, W):
    vals, idx = pl.pallas_call(
        _topk_kernel,
        grid=(GRID_STEPS,),
        in_specs=[
            pl.BlockSpec((N_TOKENS_K, EMB_DIM_K), lambda i: (0, 0)),
            pl.BlockSpec((BLOCK_ROWS, EMB_DIM_K), lambda i: (i, 0)),
            pl.BlockSpec(
                (BLOCK_ROWS, EMB_DIM_K),
                lambda i: (GRID_STEPS + jnp.minimum(i, GRID_STEPS - 2), 0),
            ),
        ],
        out_specs=[
            pl.BlockSpec((N_TOKENS_K, TOPK_K), lambda i: (0, 0)),
            pl.BlockSpec((N_TOKENS_K, TOPK_K), lambda i: (0, 0)),
        ],
        out_shape=[
            jax.ShapeDtypeStruct((N_TOKENS_K, TOPK_K), jnp.float32),
            jax.ShapeDtypeStruct((N_TOKENS_K, TOPK_K), jnp.int32),
        ],
        scratch_shapes=[
            pltpu.VMEM((NBLOCKS, N_TOKENS_K, BLOCK_ROWS), jnp.float32),
            pltpu.VMEM((NLEV, N_TOKENS_K, 128), jnp.float32),
            pltpu.VMEM((NLEV, N_TOKENS_K, 128), jnp.int32),
            pltpu.VMEM((NLEV, N_TOKENS_K, 128), jnp.float32),
            pltpu.VMEM((NLEV, N_TOKENS_K, 128), jnp.int32),
        ],
    )(x, W, W)
    return (vals, idx)
